# baseline (speedup 1.0000x reference)
import functools

import jax
import jax.numpy as jnp
from jax import lax
from jax.experimental import pallas as pl
from jax.experimental.pallas import tpu as pltpu
from jax.experimental.pallas import tpu_sc as plsc

D = 32
CTX = 10
NW = 32
CHUNK_W = 128
IDX_PER_CHUNK = CHUNK_W * CTX
DOTS_PER_WALK = CTX - 1
DOTS_PER_CHUNK = CHUNK_W * DOTS_PER_WALK
EPS = 1e-15


def _sc_dots(pos_flat, neg_flat, embedding, n_walks_half):
    n_chunks_half = n_walks_half // CHUNK_W
    chunks_per_w = n_chunks_half // NW
    n_dots = 2 * n_walks_half * DOTS_PER_WALK
    neg_out_base = n_walks_half * DOTS_PER_WALK

    mesh = plsc.VectorSubcoreMesh(core_axis_name="c", subcore_axis_name="s")

    @functools.partial(
        pl.kernel,
        mesh=mesh,
        out_type=jax.ShapeDtypeStruct((n_dots,), jnp.float32),
        compiler_params=pltpu.CompilerParams(
            needs_layout_passes=False, use_tc_tiling_on_sc=False
        ),
        scratch_types=[
            pltpu.VMEM((IDX_PER_CHUNK,), jnp.int32),
            pltpu.VMEM((IDX_PER_CHUNK,), jnp.int32),
            pltpu.VMEM((IDX_PER_CHUNK, D), jnp.float32),
            pltpu.VMEM((IDX_PER_CHUNK, D), jnp.float32),
            pltpu.VMEM((DOTS_PER_CHUNK,), jnp.float32),
            pltpu.VMEM((DOTS_PER_CHUNK,), jnp.float32),
            pltpu.SemaphoreType.DMA,
            pltpu.SemaphoreType.DMA,
            pltpu.SemaphoreType.DMA,
            pltpu.SemaphoreType.DMA,
            pltpu.SemaphoreType.DMA,
            pltpu.SemaphoreType.DMA,
        ],
    )
    def k(pos_hbm, neg_hbm, table_hbm, out_hbm,
          idx0, idx1, rows0, rows1, dots0, dots1,
          isem0, isem1, gsem0, gsem1, osem0, osem1):
        wid = lax.axis_index("s") * 2 + lax.axis_index("c")
        c0 = wid * chunks_per_w
        last = chunks_per_w - 1

        def idx_src(rw_hbm, chunk):
            return rw_hbm.at[pl.ds(chunk * IDX_PER_CHUNK, IDX_PER_CHUNK)]

        def out_dst(chunk, out_base):
            return out_hbm.at[
                pl.ds(out_base + chunk * DOTS_PER_CHUNK, DOTS_PER_CHUNK)
            ]

        def compute(rows_v, dots_v):
            def group_body(wg, carry):
                lane = jnp.arange(16, dtype=jnp.int32)
                wbase = wg * 16 + lane
                obase = wbase * DOTS_PER_WALK
                wrow = wbase * CTX
                accs = [jnp.zeros((16,), jnp.float32) for _ in range(DOTS_PER_WALK)]
                for d in range(D):
                    dvec = jnp.full((16,), d, dtype=jnp.int32)
                    s = plsc.load_gather(rows_v, [wrow, dvec])
                    for j in range(DOTS_PER_WALK):
                        r = plsc.load_gather(rows_v, [wrow + (j + 1), dvec])
                        accs[j] = accs[j] + s * r
                for j in range(DOTS_PER_WALK):
                    plsc.store_scatter(dots_v, [obase + j], accs[j])
                return carry

            lax.fori_loop(0, CHUNK_W // 16, group_body, 0)

        pltpu.async_copy(idx_src(pos_hbm, c0), idx0, isem0)
        pltpu.make_async_copy(idx_src(pos_hbm, c0), idx0, isem0).wait()
        pltpu.async_copy(table_hbm.at[idx0], rows0, gsem0)
        pltpu.async_copy(idx_src(neg_hbm, c0), idx1, isem1)

        def chunk_body(ci, carry):
            chunk = c0 + ci

            pltpu.make_async_copy(idx_src(neg_hbm, chunk), idx1, isem1).wait()
            pltpu.async_copy(table_hbm.at[idx1], rows1, gsem1)

            pltpu.make_async_copy(table_hbm.at[idx0], rows0, gsem0).wait()

            @pl.when(ci > 0)
            def _():
                pltpu.make_async_copy(dots0, out_dst(chunk - 1, 0), osem0).wait()

            compute(rows0, dots0)
            pltpu.async_copy(dots0, out_dst(chunk, 0), osem0)

            @pl.when(ci < last)
            def _():
                pltpu.async_copy(idx_src(pos_hbm, chunk + 1), idx0, isem0)
                pltpu.make_async_copy(
                    idx_src(pos_hbm, chunk + 1), idx0, isem0
                ).wait()
                pltpu.async_copy(table_hbm.at[idx0], rows0, gsem0)

            pltpu.make_async_copy(table_hbm.at[idx1], rows1, gsem1).wait()

            @pl.when(ci > 0)
            def _():
                pltpu.make_async_copy(
                    dots1, out_dst(chunk - 1, neg_out_base), osem1
                ).wait()

            compute(rows1, dots1)
            pltpu.async_copy(dots1, out_dst(chunk, neg_out_base), osem1)

            @pl.when(ci < last)
            def _():
                pltpu.async_copy(idx_src(neg_hbm, chunk + 1), idx1, isem1)

            return carry

        lax.fori_loop(0, chunks_per_w, chunk_body, 0)
        pltpu.make_async_copy(dots0, out_dst(c0 + last, 0), osem0).wait()
        pltpu.make_async_copy(dots1, out_dst(c0 + last, neg_out_base), osem1).wait()

    return k(pos_flat, neg_flat, embedding)


def _loss_from_dots(pos_d, neg_d):
    rows, cols = pos_d.shape
    blk_rows = 128
    grid = rows // blk_rows
    inv_n = 1.0 / float(pos_d.size)

    def body(pos_ref, neg_ref, out_ref):
        i = pl.program_id(0)
        sp = jax.nn.sigmoid(pos_ref[...])
        snc = jax.nn.sigmoid(-neg_ref[...])
        part = jnp.sum(-jnp.log(sp + EPS)) + jnp.sum(-jnp.log(snc + EPS))

        @pl.when(i == 0)
        def _():
            out_ref[0, 0] = 0.0

        out_ref[0, 0] += part * inv_n

    return pl.pallas_call(
        body,
        grid=(grid,),
        in_specs=[
            pl.BlockSpec((blk_rows, cols), lambda i: (i, 0)),
            pl.BlockSpec((blk_rows, cols), lambda i: (i, 0)),
        ],
        out_specs=pl.BlockSpec(memory_space=pltpu.SMEM),
        out_shape=jax.ShapeDtypeStruct((1, 1), jnp.float32),
    )(pos_d, neg_d)


def kernel(pos_rw, neg_rw, embedding):
    n = pos_rw.shape[0]
    dots = _sc_dots(pos_rw.reshape(-1), neg_rw.reshape(-1), embedding, n)
    n_half = n * DOTS_PER_WALK
    pos_d = dots[:n_half].reshape(1152, 2048)
    neg_d = dots[n_half:].reshape(1152, 2048)
    loss = _loss_from_dots(pos_d, neg_d)
    return loss[0, 0]


# speedup vs baseline: 2.6668x; 2.6668x over previous
import functools

import jax
import jax.numpy as jnp
from jax import lax
from jax.experimental import pallas as pl
from jax.experimental.pallas import tpu as pltpu
from jax.experimental.pallas import tpu_sc as plsc

D = 32
CTX = 10
NW = 32
CHUNK_W = 128
IDX_PER_CHUNK = CHUNK_W * CTX
DOTS_PER_WALK = CTX - 1
DOTS_PER_CHUNK = CHUNK_W * DOTS_PER_WALK
EPS = 1e-15


def _sc_dots(pos_flat, neg_flat, embedding, n_walks_half):
    n_chunks_half = n_walks_half // CHUNK_W
    chunks_per_w = n_chunks_half // NW
    n_dots = 2 * n_walks_half * DOTS_PER_WALK
    neg_out_base = n_walks_half * DOTS_PER_WALK

    mesh = plsc.VectorSubcoreMesh(core_axis_name="c", subcore_axis_name="s")

    @functools.partial(
        pl.kernel,
        mesh=mesh,
        out_type=jax.ShapeDtypeStruct((n_dots,), jnp.float32),
        compiler_params=pltpu.CompilerParams(
            needs_layout_passes=False, use_tc_tiling_on_sc=False
        ),
        scratch_types=[
            pltpu.VMEM((IDX_PER_CHUNK,), jnp.int32),
            pltpu.VMEM((IDX_PER_CHUNK,), jnp.int32),
            pltpu.VMEM((IDX_PER_CHUNK, D), jnp.float32),
            pltpu.VMEM((IDX_PER_CHUNK, D), jnp.float32),
            pltpu.VMEM((DOTS_PER_CHUNK,), jnp.float32),
            pltpu.VMEM((DOTS_PER_CHUNK,), jnp.float32),
            pltpu.SemaphoreType.DMA,
            pltpu.SemaphoreType.DMA,
            pltpu.SemaphoreType.DMA,
            pltpu.SemaphoreType.DMA,
            pltpu.SemaphoreType.DMA,
            pltpu.SemaphoreType.DMA,
        ],
    )
    def k(pos_hbm, neg_hbm, table_hbm, out_hbm,
          idx0, idx1, rows0, rows1, dots0, dots1,
          isem0, isem1, gsem0, gsem1, osem0, osem1):
        wid = lax.axis_index("s") * 2 + lax.axis_index("c")
        c0 = wid * chunks_per_w
        last = chunks_per_w - 1

        def idx_src(rw_hbm, chunk):
            return rw_hbm.at[pl.ds(chunk * IDX_PER_CHUNK, IDX_PER_CHUNK)]

        def out_dst(chunk, out_base):
            return out_hbm.at[
                pl.ds(out_base + chunk * DOTS_PER_CHUNK, DOTS_PER_CHUNK)
            ]

        def compute(rows_v, dots_v):
            def group_body(wg, carry):
                lane = jnp.arange(16, dtype=jnp.int32)
                wbase = wg * 16 + lane
                obase = wbase * DOTS_PER_WALK
                wrow = wbase * CTX
                accs = [jnp.zeros((16,), jnp.float32) for _ in range(DOTS_PER_WALK)]
                for d in range(D):
                    dvec = (lane + d) & (D - 1)
                    s = plsc.load_gather(rows_v, [wrow, dvec])
                    for j in range(DOTS_PER_WALK):
                        r = plsc.load_gather(rows_v, [wrow + (j + 1), dvec])
                        accs[j] = accs[j] + s * r
                for j in range(DOTS_PER_WALK):
                    plsc.store_scatter(dots_v, [obase + j], accs[j])
                return carry

            lax.fori_loop(0, CHUNK_W // 16, group_body, 0)

        pltpu.async_copy(idx_src(pos_hbm, c0), idx0, isem0)
        pltpu.make_async_copy(idx_src(pos_hbm, c0), idx0, isem0).wait()
        pltpu.async_copy(table_hbm.at[idx0], rows0, gsem0)
        pltpu.async_copy(idx_src(neg_hbm, c0), idx1, isem1)

        def chunk_body(ci, carry):
            chunk = c0 + ci

            pltpu.make_async_copy(idx_src(neg_hbm, chunk), idx1, isem1).wait()
            pltpu.async_copy(table_hbm.at[idx1], rows1, gsem1)

            pltpu.make_async_copy(table_hbm.at[idx0], rows0, gsem0).wait()

            @pl.when(ci > 0)
            def _():
                pltpu.make_async_copy(dots0, out_dst(chunk - 1, 0), osem0).wait()

            compute(rows0, dots0)
            pltpu.async_copy(dots0, out_dst(chunk, 0), osem0)

            @pl.when(ci < last)
            def _():
                pltpu.async_copy(idx_src(pos_hbm, chunk + 1), idx0, isem0)
                pltpu.make_async_copy(
                    idx_src(pos_hbm, chunk + 1), idx0, isem0
                ).wait()
                pltpu.async_copy(table_hbm.at[idx0], rows0, gsem0)

            pltpu.make_async_copy(table_hbm.at[idx1], rows1, gsem1).wait()

            @pl.when(ci > 0)
            def _():
                pltpu.make_async_copy(
                    dots1, out_dst(chunk - 1, neg_out_base), osem1
                ).wait()

            compute(rows1, dots1)
            pltpu.async_copy(dots1, out_dst(chunk, neg_out_base), osem1)

            @pl.when(ci < last)
            def _():
                pltpu.async_copy(idx_src(neg_hbm, chunk + 1), idx1, isem1)

            return carry

        lax.fori_loop(0, chunks_per_w, chunk_body, 0)
        pltpu.make_async_copy(dots0, out_dst(c0 + last, 0), osem0).wait()
        pltpu.make_async_copy(dots1, out_dst(c0 + last, neg_out_base), osem1).wait()

    return k(pos_flat, neg_flat, embedding)


def _loss_from_dots(pos_d, neg_d):
    rows, cols = pos_d.shape
    blk_rows = 128
    grid = rows // blk_rows
    inv_n = 1.0 / float(pos_d.size)

    def body(pos_ref, neg_ref, out_ref):
        i = pl.program_id(0)
        sp = jax.nn.sigmoid(pos_ref[...])
        snc = jax.nn.sigmoid(-neg_ref[...])
        part = jnp.sum(-jnp.log(sp + EPS)) + jnp.sum(-jnp.log(snc + EPS))

        @pl.when(i == 0)
        def _():
            out_ref[0, 0] = 0.0

        out_ref[0, 0] += part * inv_n

    return pl.pallas_call(
        body,
        grid=(grid,),
        in_specs=[
            pl.BlockSpec((blk_rows, cols), lambda i: (i, 0)),
            pl.BlockSpec((blk_rows, cols), lambda i: (i, 0)),
        ],
        out_specs=pl.BlockSpec(memory_space=pltpu.SMEM),
        out_shape=jax.ShapeDtypeStruct((1, 1), jnp.float32),
    )(pos_d, neg_d)


def kernel(pos_rw, neg_rw, embedding):
    n = pos_rw.shape[0]
    dots = _sc_dots(pos_rw.reshape(-1), neg_rw.reshape(-1), embedding, n)
    n_half = n * DOTS_PER_WALK
    pos_d = dots[:n_half].reshape(1152, 2048)
    neg_d = dots[n_half:].reshape(1152, 2048)
    loss = _loss_from_dots(pos_d, neg_d)
    return loss[0, 0]
